# baseline (speedup 1.0000x reference)
import dataclasses

import jax
from jax import lax
import jax.numpy as jnp
from jax.experimental import pallas as pl
from jax.experimental.pallas import tpu as pltpu
from jax.experimental.pallas import tpu_sc as plsc

LANES = 16
CL = 32
NBUF = 4
NCS = 8


def _sc_gather_add(xs, times_lb, pe):
    L, B, dim = xs.shape
    n_pe = pe.shape[0]
    csw = dim // NCS
    n_workers = 32
    ngr = n_workers // NCS
    lwl = L // ngr
    nc = lwl // CL
    cb = CL * B

    mesh = plsc.VectorSubcoreMesh(core_axis_name="core", subcore_axis_name="subcore")

    scratch = (
        [pltpu.VMEM((n_pe, csw), jnp.float32)]
        + [pltpu.VMEM((lwl * B,), jnp.int32)]
        + [pltpu.VMEM((CL, B, csw), jnp.float32) for _ in range(NBUF)]
        + [pltpu.SemaphoreType.DMA for _ in range(2 * NBUF)]
    )

    cp = pltpu.CompilerParams()
    if "needs_layout_passes" in pltpu.CompilerParams.__dataclass_fields__:
        cp = dataclasses.replace(cp, needs_layout_passes=False)

    @pl.kernel(
        out_type=jax.ShapeDtypeStruct((L, B, dim), jnp.float32),
        mesh=mesh,
        scratch_types=scratch,
        compiler_params=cp,
    )
    def k(xs_hbm, t_hbm, pe_hbm, o_hbm, pe_sl, idx_v, *bufs):
        ab = bufs[0:NBUF]
        sx = bufs[NBUF:2 * NBUF]
        so = bufs[2 * NBUF:3 * NBUF]

        wid = lax.axis_index("core") * 16 + lax.axis_index("subcore")
        cs = wid // ngr
        g = wid % ngr
        l_base = g * lwl
        c0 = cs * csw

        pltpu.sync_copy(pe_hbm.at[:, pl.ds(c0, csw)], pe_sl)
        pltpu.sync_copy(t_hbm.at[pl.ds(l_base * B, lwl * B)], idx_v)

        def issue_loads(c, j):
            l0 = l_base + c * CL
            pltpu.async_copy(xs_hbm.at[pl.ds(l0, CL), :, pl.ds(c0, csw)], ab[j], sx[j])

        def wait_loads(c, j):
            l0 = l_base + c * CL
            pltpu.make_async_copy(
                xs_hbm.at[pl.ds(l0, CL), :, pl.ds(c0, csw)], ab[j], sx[j]).wait()

        def wait_store(c, j):
            l0 = l_base + c * CL
            pltpu.make_async_copy(
                ab[j], o_hbm.at[pl.ds(l0, CL), :, pl.ds(c0, csw)], so[j]).wait()

        for j in range(2):
            issue_loads(j, j)

        @pl.loop(0, nc, step=NBUF)
        def _(cbase):
            for j in range(NBUF):
                c = cbase + j
                wait_loads(c, j)

                @pl.loop(0, CL // 4)
                def _(lr4):
                    tv = idx_v[pl.ds(c * cb + lr4 * 16, 16)]
                    for i in range(16):
                        t = tv[i]
                        lr = lr4 * 4 + i // B
                        br = i % B
                        for cc in range(0, csw, LANES):
                            plsc.addupdate(
                                ab[j].at[lr, br, pl.ds(cc, LANES)],
                                pe_sl[t, pl.ds(cc, LANES)],
                            )

                l0 = l_base + c * CL
                pltpu.async_copy(ab[j], o_hbm.at[pl.ds(l0, CL), :, pl.ds(c0, csw)], so[j])

                @pl.when(c + 2 < nc)
                def _():
                    @pl.when(c >= 2)
                    def _():
                        wait_store(c - 2, (j + 2) % NBUF)
                    issue_loads(c + 2, (j + 2) % NBUF)

        for j in range(NBUF):
            wait_store(nc - NBUF + j, j)

    return k(xs, times_lb, pe)


def kernel(xs, times, pe):
    L, B, dim = xs.shape
    times_lb = times.astype(jnp.int32).T.reshape(L * B)
    return _sc_gather_add(xs, times_lb, pe)


# speedup vs baseline: 1.4751x; 1.4751x over previous
import jax
from jax import lax
import jax.numpy as jnp
from jax.experimental import pallas as pl
from jax.experimental.pallas import tpu as pltpu
from jax.experimental.pallas import tpu_sc as plsc

LANES = 16
CH = 16
NBUF = 2


def _sc_gather_add(xs, times_flat, pe):
    L, B, dim = xs.shape
    n_workers = 32
    lw = (L * B) // n_workers
    nc = lw // CH
    wpb = n_workers // B

    mesh = plsc.VectorSubcoreMesh(core_axis_name="core", subcore_axis_name="subcore")

    scratch = (
        [pltpu.VMEM((lw,), jnp.int32)]
        + [pltpu.VMEM((CH, dim), jnp.float32) for _ in range(3 * NBUF)]
        + [pltpu.SemaphoreType.DMA for _ in range(3 * NBUF)]
    )

    @pl.kernel(
        out_type=jax.ShapeDtypeStruct((L, B, dim), jnp.float32),
        mesh=mesh,
        scratch_types=scratch,
    )
    def k(xs_hbm, t_hbm, pe_hbm, o_hbm, idx_v,
          xb0, xb1, pb0, pb1, ob0, ob1,
          sx0, sx1, sp0, sp1, so0, so1):
        xb = (xb0, xb1)
        pb = (pb0, pb1)
        ob = (ob0, ob1)
        sx = (sx0, sx1)
        sp = (sp0, sp1)
        so = (so0, so1)

        wid = lax.axis_index("core") * 16 + lax.axis_index("subcore")
        b = wid // wpb
        l_base = (wid % wpb) * lw

        pltpu.sync_copy(t_hbm.at[pl.ds(b * L + l_base, lw)], idx_v)

        def issue_loads(c, j):
            l0 = l_base + c * CH
            pltpu.async_copy(xs_hbm.at[pl.ds(l0, CH), b, :], xb[j], sx[j])
            pltpu.async_copy(pe_hbm.at[idx_v.at[pl.ds(c * CH, CH)]], pb[j], sp[j])

        def wait_loads(c, j):
            l0 = l_base + c * CH
            pltpu.make_async_copy(xs_hbm.at[pl.ds(l0, CH), b, :], xb[j], sx[j]).wait()
            pltpu.make_async_copy(
                pe_hbm.at[idx_v.at[pl.ds(c * CH, CH)]], pb[j], sp[j]).wait()

        def wait_store(c, j):
            l0 = l_base + c * CH
            pltpu.make_async_copy(ob[j], o_hbm.at[pl.ds(l0, CH), b, :], so[j]).wait()

        for j in range(NBUF):
            issue_loads(j, j)

        @pl.loop(0, nc, step=NBUF)
        def _(cbase):
            for j in range(NBUF):
                c = cbase + j
                wait_loads(c, j)

                @pl.when(c >= NBUF)
                def _():
                    wait_store(c - NBUF, j)

                @pl.loop(0, CH)
                def _(r):
                    for cc in range(0, dim, LANES):
                        ob[j][r, pl.ds(cc, LANES)] = (
                            xb[j][r, pl.ds(cc, LANES)] + pb[j][r, pl.ds(cc, LANES)]
                        )

                @pl.when(c + NBUF < nc)
                def _():
                    issue_loads(c + NBUF, j)

                l0 = l_base + c * CH
                pltpu.async_copy(ob[j], o_hbm.at[pl.ds(l0, CH), b, :], so[j])

        for j in range(NBUF):
            wait_store(nc - NBUF + j, j)

    return k(xs, times_flat, pe)


def kernel(xs, times, pe):
    L, B, dim = xs.shape
    times_flat = times.astype(jnp.int32).reshape(B * L)
    return _sc_gather_add(xs, times_flat, pe)
